# Initial kernel scaffold; baseline (speedup 1.0000x reference)
import functools

import jax
import jax.numpy as jnp
from jax import lax
from jax.experimental import pallas as pl
from jax.experimental.pallas import tpu as pltpu
from jax.experimental.pallas import tpu_sc as plsc

NC = 2
NS = 16
NW = NC * NS
L = 16


def _make_sg_kernel(B, K, D, V):
    PER_W = B // NW
    G = 64
    CHUNKS = PER_W // G
    NIR = G * K // 128

    mesh = plsc.VectorSubcoreMesh(
        core_axis_name="c", subcore_axis_name="s",
        num_cores=NC, num_subcores=NS)

    @functools.partial(
        pl.kernel,
        out_type=jax.ShapeDtypeStruct((B, 1 + K), jnp.float32),
        mesh=mesh,
        scratch_types=[
            pltpu.VMEM((G,), jnp.int32),
            pltpu.VMEM((G,), jnp.int32),
            pltpu.VMEM((NIR, 128), jnp.int32),
            pltpu.VMEM((G, D), jnp.float32),
            pltpu.VMEM((G, D), jnp.float32),
            pltpu.VMEM((G * K, D), jnp.float32),
            pltpu.VMEM((G, 1 + K), jnp.float32),
            pltpu.SemaphoreType.DMA,
        ],
    )
    def sg(tgt_hbm, ctx_hbm, neg_hbm, w_hbm, c_hbm, out_hbm,
           ti_v, ci_v, ni_v, tr_v, cr_v, nr_v, ov_v, sem):
        wid = lax.axis_index("s") * NC + lax.axis_index("c")
        elem0 = wid * PER_W

        def chunk_body(ci, carry):
            base = elem0 + ci * G
            pltpu.sync_copy(tgt_hbm.at[pl.ds(base, G)], ti_v)
            pltpu.sync_copy(ctx_hbm.at[pl.ds(base, G)], ci_v)
            nrow0 = base * K // 128
            pltpu.sync_copy(neg_hbm.at[pl.ds(nrow0, NIR)], ni_v)
            copies = [
                pltpu.async_copy(w_hbm.at[ti_v], tr_v, sem),
                pltpu.async_copy(c_hbm.at[ci_v], cr_v, sem),
            ]
            for j in range(NIR):
                copies.append(pltpu.async_copy(
                    c_hbm.at[ni_v.at[j]], nr_v.at[pl.ds(j * 128, 128)], sem))
            for cp in copies:
                cp.wait()

            for g in range(G // L):
                rows16 = lax.iota(jnp.int32, L) + g * L
                nbase = rows16 * K
                zero = jnp.zeros((L,), jnp.float32)

                def dbody(d, acc):
                    pos, negs = acc
                    dv = jnp.full((L,), d, jnp.int32)
                    t = plsc.load_gather(tr_v, [rows16, dv])
                    cv = plsc.load_gather(cr_v, [rows16, dv])
                    pos = pos + t * cv
                    negs = tuple(
                        negs[k] + t * plsc.load_gather(nr_v, [nbase + k, dv])
                        for k in range(K))
                    return (pos, negs)

                pos, negs = lax.fori_loop(0, D, dbody, (zero, (zero,) * K))
                plsc.store_scatter(
                    ov_v, [rows16, jnp.zeros((L,), jnp.int32)], pos)
                for k in range(K):
                    plsc.store_scatter(
                        ov_v, [rows16, jnp.full((L,), k + 1, jnp.int32)],
                        negs[k])
            pltpu.sync_copy(ov_v, out_hbm.at[pl.ds(base, G)])
            return carry

        lax.fori_loop(0, CHUNKS, chunk_body, 0)

    return sg


def kernel(target, context, neg_samples, W, C):
    B = target.shape[0]
    K = neg_samples.shape[1]
    V, D = W.shape
    tgt = target.astype(jnp.int32)
    ctx = context.astype(jnp.int32)
    neg = neg_samples.astype(jnp.int32).reshape(B * K // 128, 128)
    logits = _make_sg_kernel(B, K, D, V)(tgt, ctx, neg, W, C)
    labels = jnp.concatenate(
        [jnp.ones((B, 1), jnp.float32), jnp.zeros((B, K), jnp.float32)],
        axis=1)
    return (logits, labels)


# speedup vs baseline: 3.9602x; 3.9602x over previous
import functools

import jax
import jax.numpy as jnp
from jax import lax
from jax.experimental import pallas as pl
from jax.experimental.pallas import tpu as pltpu
from jax.experimental.pallas import tpu_sc as plsc

NC = 2
NS = 16
NW = NC * NS
L = 16


def _make_sg_kernel(B, K, D, V):
    PER_W = B // NW
    G = 64
    CHUNKS = PER_W // G
    NIR = G * K // 128

    mesh = plsc.VectorSubcoreMesh(
        core_axis_name="c", subcore_axis_name="s",
        num_cores=NC, num_subcores=NS)

    @functools.partial(
        pl.kernel,
        out_type=jax.ShapeDtypeStruct((B, 1 + K), jnp.float32),
        mesh=mesh,
        scratch_types=[
            pltpu.VMEM((G,), jnp.int32),
            pltpu.VMEM((G,), jnp.int32),
            pltpu.VMEM((G * K,), jnp.int32),
            pltpu.VMEM((G, D), jnp.float32),
            pltpu.VMEM((G, D), jnp.float32),
            pltpu.VMEM((G * K, D), jnp.float32),
            pltpu.VMEM((G, 1 + K), jnp.float32),
            pltpu.SemaphoreType.DMA,
        ],
        compiler_params=pltpu.CompilerParams(
            needs_layout_passes=False, use_tc_tiling_on_sc=False),
    )
    def sg(tgt_hbm, ctx_hbm, neg_hbm, w_hbm, c_hbm, out_hbm,
           ti_v, ci_v, ni_v, tr_v, cr_v, nr_v, ov_v, sem):
        wid = lax.axis_index("s") * NC + lax.axis_index("c")
        elem0 = wid * PER_W

        def chunk_body(ci, carry):
            base = elem0 + ci * G
            pltpu.sync_copy(tgt_hbm.at[pl.ds(base, G)], ti_v)
            pltpu.sync_copy(ctx_hbm.at[pl.ds(base, G)], ci_v)
            pltpu.sync_copy(neg_hbm.at[pl.ds(base * K, G * K)], ni_v)
            copies = [
                pltpu.async_copy(w_hbm.at[ti_v], tr_v, sem),
                pltpu.async_copy(c_hbm.at[ci_v], cr_v, sem),
            ]
            for j in range(NIR):
                copies.append(pltpu.async_copy(
                    c_hbm.at[ni_v.at[pl.ds(j * 128, 128)]],
                    nr_v.at[pl.ds(j * 128, 128)], sem))
            for cp in copies:
                cp.wait()

            for g in range(G // L):
                rows16 = lax.iota(jnp.int32, L) + g * L
                nbase = rows16 * K
                zero = jnp.zeros((L,), jnp.float32)

                def dbody(d, acc):
                    pos, negs = acc
                    dv = jnp.full((L,), d, jnp.int32)
                    t = plsc.load_gather(tr_v, [rows16, dv])
                    cv = plsc.load_gather(cr_v, [rows16, dv])
                    pos = pos + t * cv
                    negs = tuple(
                        negs[k] + t * plsc.load_gather(nr_v, [nbase + k, dv])
                        for k in range(K))
                    return (pos, negs)

                pos, negs = lax.fori_loop(0, D, dbody, (zero, (zero,) * K))
                plsc.store_scatter(
                    ov_v, [rows16, jnp.zeros((L,), jnp.int32)], pos)
                for k in range(K):
                    plsc.store_scatter(
                        ov_v, [rows16, jnp.full((L,), k + 1, jnp.int32)],
                        negs[k])
            pltpu.sync_copy(ov_v, out_hbm.at[pl.ds(base, G)])
            return carry

        lax.fori_loop(0, CHUNKS, chunk_body, 0)

    return sg


def kernel(target, context, neg_samples, W, C):
    B = target.shape[0]
    K = neg_samples.shape[1]
    V, D = W.shape
    tgt = target.astype(jnp.int32)
    ctx = context.astype(jnp.int32)
    neg = neg_samples.astype(jnp.int32).reshape(B * K)
    logits = _make_sg_kernel(B, K, D, V)(tgt, ctx, neg, W, C)
    labels = jnp.concatenate(
        [jnp.ones((B, 1), jnp.float32), jnp.zeros((B, K), jnp.float32)],
        axis=1)
    return (logits, labels)


# speedup vs baseline: 5.8903x; 1.4874x over previous
import functools

import jax
import jax.numpy as jnp
from jax import lax
from jax.experimental import pallas as pl
from jax.experimental.pallas import tpu as pltpu
from jax.experimental.pallas import tpu_sc as plsc

NC = 2
NS = 16
NW = NC * NS
L = 16
BR = 2048


def _fuse_tables(wt, ct):
    d, v = wt.shape

    def body(w_ref, c_ref, o_ref):
        z = jnp.concatenate([w_ref[...], c_ref[...]], axis=0)
        o_ref[...] = z.T

    return pl.pallas_call(
        body,
        grid=(pl.cdiv(v, BR),),
        in_specs=[pl.BlockSpec((d, BR), lambda i: (0, i)),
                  pl.BlockSpec((d, BR), lambda i: (0, i))],
        out_specs=pl.BlockSpec((BR, 2 * d), lambda i: (i, 0)),
        out_shape=jax.ShapeDtypeStruct((v, 2 * d), jnp.float32),
    )(wt, ct)


def _make_sg_kernel(B, K, D, V):
    PER_W = B // NW
    G = 32
    CHUNKS = PER_W // G
    NIR = G * K // 128

    mesh = plsc.VectorSubcoreMesh(
        core_axis_name="c", subcore_axis_name="s",
        num_cores=NC, num_subcores=NS)

    @functools.partial(
        pl.kernel,
        out_type=jax.ShapeDtypeStruct((B, 2 * D), jnp.float32),
        mesh=mesh,
        scratch_types=[
            pltpu.VMEM((G,), jnp.int32),
            pltpu.VMEM((G,), jnp.int32),
            pltpu.VMEM((G * K,), jnp.int32),
            pltpu.VMEM((G, 2 * D), jnp.float32),
            pltpu.VMEM((G, 2 * D), jnp.float32),
            pltpu.VMEM((G * K, 2 * D), jnp.float32),
            pltpu.VMEM((G, 2 * D), jnp.float32),
            pltpu.SemaphoreType.DMA,
        ],
        compiler_params=pltpu.CompilerParams(
            needs_layout_passes=False, use_tc_tiling_on_sc=True),
    )
    def sg(tgt_hbm, ctx_hbm, neg_hbm, f_hbm, out_hbm,
           ti_v, ci_v, ni_v, tr_v, cr_v, nr_v, ov_v, sem):
        wid = lax.axis_index("s") * NC + lax.axis_index("c")
        elem0 = wid * PER_W

        def chunk_body(ci, carry):
            base = elem0 + ci * G
            pltpu.sync_copy(tgt_hbm.at[pl.ds(base, G)], ti_v)
            pltpu.sync_copy(ctx_hbm.at[pl.ds(base, G)], ci_v)
            pltpu.sync_copy(neg_hbm.at[pl.ds(base * K, G * K)], ni_v)
            copies = [
                pltpu.async_copy(f_hbm.at[ti_v], tr_v, sem),
                pltpu.async_copy(f_hbm.at[ci_v], cr_v, sem),
            ]
            for j in range(NIR):
                copies.append(pltpu.async_copy(
                    f_hbm.at[ni_v.at[pl.ds(j * 128, 128)]],
                    nr_v.at[pl.ds(j * 128, 128)], sem))
            for cp in copies:
                cp.wait()

            for g in range(G // L):
                rows16 = lax.iota(jnp.int32, L) + g * L
                nbase = rows16 * K
                zero = jnp.zeros((L,), jnp.float32)

                def dbody(d, acc):
                    pos, negs = acc
                    dv = jnp.full((L,), d, jnp.int32)
                    dv64 = dv + D
                    t = plsc.load_gather(tr_v, [rows16, dv])
                    cv = plsc.load_gather(cr_v, [rows16, dv64])
                    pos = pos + t * cv
                    negs = tuple(
                        negs[k] + t * plsc.load_gather(nr_v, [nbase + k, dv64])
                        for k in range(K))
                    return (pos, negs)

                pos, negs = lax.fori_loop(0, D, dbody, (zero, (zero,) * K))
                plsc.store_scatter(
                    ov_v, [rows16, jnp.zeros((L,), jnp.int32)], pos)
                for k in range(K):
                    plsc.store_scatter(
                        ov_v, [rows16, jnp.full((L,), k + 1, jnp.int32)],
                        negs[k])
            pltpu.sync_copy(ov_v, out_hbm.at[pl.ds(base, G)])
            return carry

        lax.fori_loop(0, CHUNKS, chunk_body, 0)

    return sg


def kernel(target, context, neg_samples, W, C):
    B = target.shape[0]
    K = neg_samples.shape[1]
    V, D = W.shape
    tgt = target.astype(jnp.int32)
    ctx = context.astype(jnp.int32)
    neg = neg_samples.astype(jnp.int32).reshape(B * K)
    fused = _fuse_tables(W.T, C.T)
    scores = _make_sg_kernel(B, K, D, V)(tgt, ctx, neg, fused)
    logits = scores[:, :1 + K]
    labels = jnp.concatenate(
        [jnp.ones((B, 1), jnp.float32), jnp.zeros((B, K), jnp.float32)],
        axis=1)
    return (logits, labels)


# speedup vs baseline: 6.3280x; 1.0743x over previous
import functools

import jax
import jax.numpy as jnp
from jax import lax
from jax.experimental import pallas as pl
from jax.experimental.pallas import tpu as pltpu
from jax.experimental.pallas import tpu_sc as plsc

NC = 2
NS = 16
NW = NC * NS
L = 16
BR = 2048


def _fuse_tables(wt, ct):
    d, v = wt.shape

    def body(w_ref, c_ref, o_ref):
        z = jnp.concatenate([w_ref[...], c_ref[...]], axis=0)
        o_ref[...] = z.T

    return pl.pallas_call(
        body,
        grid=(pl.cdiv(v, BR),),
        in_specs=[pl.BlockSpec((d, BR), lambda i: (0, i)),
                  pl.BlockSpec((d, BR), lambda i: (0, i))],
        out_specs=pl.BlockSpec((BR, 2 * d), lambda i: (i, 0)),
        out_shape=jax.ShapeDtypeStruct((v, 2 * d), jnp.float32),
    )(wt, ct)


def _make_sg_kernel(B, K, D, V):
    PER_W = B // NW
    G = 16
    CHUNKS = PER_W // G
    PAIRS = CHUNKS // 2
    GK = G * K

    mesh = plsc.VectorSubcoreMesh(
        core_axis_name="c", subcore_axis_name="s",
        num_cores=NC, num_subcores=NS)

    @functools.partial(
        pl.kernel,
        out_type=jax.ShapeDtypeStruct((B, 2 * D), jnp.float32),
        mesh=mesh,
        scratch_types=[
            pltpu.VMEM((PER_W,), jnp.int32),
            pltpu.VMEM((PER_W,), jnp.int32),
            pltpu.VMEM((PER_W * K,), jnp.int32),
            [pltpu.VMEM((G, 2 * D), jnp.float32)] * 2,
            [pltpu.VMEM((G, 2 * D), jnp.float32)] * 2,
            [pltpu.VMEM((GK, 2 * D), jnp.float32)] * 2,
            pltpu.VMEM((G, 2 * D), jnp.float32),
            [pltpu.SemaphoreType.DMA] * 2,
        ],
        compiler_params=pltpu.CompilerParams(
            needs_layout_passes=False, use_tc_tiling_on_sc=True),
    )
    def sg(tgt_hbm, ctx_hbm, neg_hbm, f_hbm, out_hbm,
           ti_v, ci_v, ni_v, tr_v, cr_v, nr_v, ov_v, sems):
        wid = lax.axis_index("s") * NC + lax.axis_index("c")
        elem0 = wid * PER_W

        pltpu.sync_copy(tgt_hbm.at[pl.ds(elem0, PER_W)], ti_v)
        pltpu.sync_copy(ctx_hbm.at[pl.ds(elem0, PER_W)], ci_v)
        pltpu.sync_copy(neg_hbm.at[pl.ds(elem0 * K, PER_W * K)], ni_v)

        def descriptors(s, c):
            pairs = [
                (f_hbm.at[ti_v.at[pl.ds(c * G, G)]], tr_v[s]),
                (f_hbm.at[ci_v.at[pl.ds(c * G, G)]], cr_v[s]),
            ]
            for j in range(0, GK, 128):
                n = min(128, GK - j)
                pairs.append((f_hbm.at[ni_v.at[pl.ds(c * GK + j, n)]],
                              nr_v[s].at[pl.ds(j, n)]))
            return pairs

        def fire(s, c):
            for src, dst in descriptors(s, c):
                pltpu.async_copy(src, dst, sems[s])

        def drain(s, c):
            for src, dst in descriptors(s, c):
                pltpu.make_async_copy(src, dst, sems[s]).wait()

        def compute(s, c):
            base = elem0 + c * G
            rows16 = lax.iota(jnp.int32, L)
            nbase = rows16 * K
            zero = jnp.zeros((L,), jnp.float32)

            def dbody(d, acc):
                pos, negs = acc
                dv = jnp.full((L,), d, jnp.int32)
                dv64 = dv + D
                t = plsc.load_gather(tr_v[s], [rows16, dv])
                cv = plsc.load_gather(cr_v[s], [rows16, dv64])
                pos = pos + t * cv
                negs = tuple(
                    negs[k] + t * plsc.load_gather(nr_v[s], [nbase + k, dv64])
                    for k in range(K))
                return (pos, negs)

            pos, negs = lax.fori_loop(0, D, dbody, (zero, (zero,) * K))
            plsc.store_scatter(
                ov_v, [rows16, jnp.zeros((L,), jnp.int32)], pos)
            for k in range(K):
                plsc.store_scatter(
                    ov_v, [rows16, jnp.full((L,), k + 1, jnp.int32)],
                    negs[k])
            pltpu.sync_copy(ov_v, out_hbm.at[pl.ds(base, G)])

        fire(0, 0)
        fire(1, 1)

        def pair_body(i, carry):
            c0 = 2 * i
            drain(0, c0)
            compute(0, c0)

            @pl.when(i + 1 < PAIRS)
            def _():
                fire(0, c0 + 2)

            drain(1, c0 + 1)
            compute(1, c0 + 1)

            @pl.when(i + 1 < PAIRS)
            def _():
                fire(1, c0 + 3)

            return carry

        lax.fori_loop(0, PAIRS, pair_body, 0)

    return sg


def kernel(target, context, neg_samples, W, C):
    B = target.shape[0]
    K = neg_samples.shape[1]
    V, D = W.shape
    tgt = target.astype(jnp.int32)
    ctx = context.astype(jnp.int32)
    neg = neg_samples.astype(jnp.int32).reshape(B * K)
    fused = _fuse_tables(W.T, C.T)
    scores = _make_sg_kernel(B, K, D, V)(tgt, ctx, neg, fused)
    logits = scores[:, :1 + K]
    labels = jnp.concatenate(
        [jnp.ones((B, 1), jnp.float32), jnp.zeros((B, K), jnp.float32)],
        axis=1)
    return (logits, labels)


# speedup vs baseline: 6.4232x; 1.0150x over previous
import functools

import jax
import jax.numpy as jnp
from jax import lax
from jax.experimental import pallas as pl
from jax.experimental.pallas import tpu as pltpu
from jax.experimental.pallas import tpu_sc as plsc

NC = 2
NS = 16
NW = NC * NS
L = 16
BR = 2048


def _fuse_tables(wt, ct):
    d, v = wt.shape

    def body(w_ref, c_ref, o_ref):
        z = jnp.concatenate([w_ref[...], c_ref[...]], axis=0)
        o_ref[...] = z.T

    return pl.pallas_call(
        body,
        grid=(pl.cdiv(v, BR),),
        in_specs=[pl.BlockSpec((d, BR), lambda i: (0, i)),
                  pl.BlockSpec((d, BR), lambda i: (0, i))],
        out_specs=pl.BlockSpec((BR, 2 * d), lambda i: (i, 0)),
        out_shape=jax.ShapeDtypeStruct((v, 2 * d), jnp.float32),
    )(wt, ct)


def _make_sg_kernel(B, K, D, V):
    PER_W = B // NW
    G = 16
    CHUNKS = PER_W // G
    PAIRS = CHUNKS // 2
    GK = G * K

    mesh = plsc.VectorSubcoreMesh(
        core_axis_name="c", subcore_axis_name="s",
        num_cores=NC, num_subcores=NS)

    @functools.partial(
        pl.kernel,
        out_type=jax.ShapeDtypeStruct((B, 2 * D), jnp.float32),
        mesh=mesh,
        scratch_types=[
            pltpu.VMEM((PER_W,), jnp.int32),
            pltpu.VMEM((PER_W,), jnp.int32),
            pltpu.VMEM((PER_W * K,), jnp.int32),
            [pltpu.VMEM((G, 2 * D), jnp.float32)] * 2,
            [pltpu.VMEM((G, 2 * D), jnp.float32)] * 2,
            [pltpu.VMEM((GK, 2 * D), jnp.float32)] * 2,
            pltpu.VMEM((G, 2 * D), jnp.float32),
            [pltpu.SemaphoreType.DMA] * 2,
        ],
        compiler_params=pltpu.CompilerParams(
            needs_layout_passes=False, use_tc_tiling_on_sc=True),
    )
    def sg(tgt_hbm, ctx_hbm, neg_hbm, f_hbm, out_hbm,
           ti_v, ci_v, ni_v, tr_v, cr_v, nr_v, ov_v, sems):
        wid = lax.axis_index("s") * NC + lax.axis_index("c")
        elem0 = wid * PER_W

        pltpu.sync_copy(tgt_hbm.at[pl.ds(elem0, PER_W)], ti_v)
        pltpu.sync_copy(ctx_hbm.at[pl.ds(elem0, PER_W)], ci_v)
        pltpu.sync_copy(neg_hbm.at[pl.ds(elem0 * K, PER_W * K)], ni_v)

        def descriptors(s, c):
            pairs = [
                (f_hbm.at[ti_v.at[pl.ds(c * G, G)]], tr_v[s]),
                (f_hbm.at[ci_v.at[pl.ds(c * G, G)]], cr_v[s]),
            ]
            for j in range(0, GK, 128):
                n = min(128, GK - j)
                pairs.append((f_hbm.at[ni_v.at[pl.ds(c * GK + j, n)]],
                              nr_v[s].at[pl.ds(j, n)]))
            return pairs

        def fire(s, c):
            for src, dst in descriptors(s, c):
                pltpu.async_copy(src, dst, sems[s])

        def drain(s, c):
            for src, dst in descriptors(s, c):
                pltpu.make_async_copy(src, dst, sems[s]).wait()

        def compute(s, c):
            base = elem0 + c * G
            rows16 = lax.iota(jnp.int32, L)
            nbase = rows16 * K
            zero = jnp.zeros((L,), jnp.float32)

            UNROLL = 4

            def dbody(i, acc):
                pos, negs = acc
                d0 = i * UNROLL
                for u in range(UNROLL):
                    dv = jnp.full((L,), d0 + u, jnp.int32)
                    dv64 = dv + D
                    t = plsc.load_gather(tr_v[s], [rows16, dv])
                    cv = plsc.load_gather(cr_v[s], [rows16, dv64])
                    pos = pos + t * cv
                    negs = tuple(
                        negs[k]
                        + t * plsc.load_gather(nr_v[s], [nbase + k, dv64])
                        for k in range(K))
                return (pos, negs)

            pos, negs = lax.fori_loop(
                0, D // UNROLL, dbody, (zero, (zero,) * K))
            plsc.store_scatter(
                ov_v, [rows16, jnp.zeros((L,), jnp.int32)], pos)
            for k in range(K):
                plsc.store_scatter(
                    ov_v, [rows16, jnp.full((L,), k + 1, jnp.int32)],
                    negs[k])
            pltpu.sync_copy(ov_v, out_hbm.at[pl.ds(base, G)])

        fire(0, 0)
        fire(1, 1)

        def pair_body(i, carry):
            c0 = 2 * i
            drain(0, c0)
            compute(0, c0)

            @pl.when(i + 1 < PAIRS)
            def _():
                fire(0, c0 + 2)

            drain(1, c0 + 1)
            compute(1, c0 + 1)

            @pl.when(i + 1 < PAIRS)
            def _():
                fire(1, c0 + 3)

            return carry

        lax.fori_loop(0, PAIRS, pair_body, 0)

    return sg


def kernel(target, context, neg_samples, W, C):
    B = target.shape[0]
    K = neg_samples.shape[1]
    V, D = W.shape
    tgt = target.astype(jnp.int32)
    ctx = context.astype(jnp.int32)
    neg = neg_samples.astype(jnp.int32).reshape(B * K)
    fused = _fuse_tables(W.T, C.T)
    scores = _make_sg_kernel(B, K, D, V)(tgt, ctx, neg, fused)
    logits = scores[:, :1 + K]
    labels = jnp.concatenate(
        [jnp.ones((B, 1), jnp.float32), jnp.zeros((B, K), jnp.float32)],
        axis=1)
    return (logits, labels)
